# baseline (speedup 1.0000x reference)
import jax
import jax.numpy as jnp
from jax import lax
from jax.experimental import pallas as pl
from jax.experimental.pallas import tpu as pltpu
from jax.experimental.pallas import tpu_sc as plsc

SIZE = 51
NUM_ACT = 96
ROWS = 224
HALF = 112
NSLAB = 8 * NUM_ACT
NC, NS = 2, 16
NW = NC * NS
SPW = NSLAB // NW
CSTRIDE = 56
TABLE_PAD = NUM_ACT * CSTRIDE


def _body(x_hbm, ctab_hbm, dtab_hbm, invg_hbm, out_hbm,
          ctab_v, dtab_v, invg_v, b0, b1, b2, b3,
          isem0, isem1, isem2, isem3, osem0, osem1, osem2, osem3):
    wid = lax.axis_index("s") * NC + lax.axis_index("c")
    pltpu.sync_copy(ctab_hbm, ctab_v)
    pltpu.sync_copy(dtab_hbm, dtab_v)
    pltpu.sync_copy(invg_hbm, invg_v)
    invg = invg_v[...]
    chunk0 = wid * SPW * 2

    def src(i):
        return x_hbm.at[chunk0 // 2 + lax.div(i, 2),
                        pl.ds(lax.rem(i, 2) * HALF, HALF), :]

    def dst(i):
        return out_hbm.at[chunk0 // 2 + lax.div(i, 2),
                          pl.ds(lax.rem(i, 2) * HALF, HALF), :]

    def start_in(i, buf, sem):
        pltpu.async_copy(src(i), buf, sem)

    def wait_in(buf, sem):
        pltpu.make_async_copy(x_hbm.at[0, pl.ds(0, HALF), :], buf, sem).wait()

    def start_out(i, buf, sem):
        pltpu.async_copy(buf, dst(i), sem)

    def wait_out(buf, sem):
        pltpu.make_async_copy(buf, out_hbm.at[0, pl.ds(0, HALF), :], sem).wait()

    def make_base(i):
        c = lax.rem(chunk0 // 2 + lax.div(i, 2), NUM_ACT)
        return c * CSTRIDE

    def compute(buf, base):
        ctab_sl = ctab_v.at[pl.ds(base, CSTRIDE)]
        dtab_sl = dtab_v.at[pl.ds(base, CSTRIDE)]

        @plsc.parallel_loop(0, HALF, step=1, unroll=2)
        def _(r):
            for k in range(ROWS // 16):
                v = buf[r, pl.ds(k * 16, 16)]
                t = v * invg + jnp.float32(25.0)
                t = jnp.minimum(jnp.maximum(t, jnp.float32(0.0)),
                                jnp.float32(49.0))
                j = t.astype(jnp.int32)
                frac = t - j.astype(jnp.float32)
                cv = plsc.load_gather(ctab_sl, [j])
                dv = plsc.load_gather(dtab_sl, [j])
                buf[r, pl.ds(k * 16, 16)] = cv + frac * dv

    ring = ((b0, isem0, osem0), (b1, isem1, osem1),
            (b2, isem2, osem2), (b3, isem3, osem3))

    start_in(jnp.int32(0), b0, isem0)
    start_in(jnp.int32(1), b1, isem1)
    for b in range(2):
        buf, isem, osem = ring[b]
        i = jnp.int32(b)
        wait_in(buf, isem)
        compute(buf, make_base(i))
        start_out(i, buf, osem)
        start_in(i + 2, ring[b + 2][0], ring[b + 2][1])

    def loop_body(g, carry):
        i0 = g * 4
        for b in range(4):
            buf, isem, osem = ring[(2 + b) % 4]
            i = i0 + jnp.int32(b - 2)
            wait_in(buf, isem)
            compute(buf, make_base(i))
            start_out(i, buf, osem)
            nbuf, nisem, nosem = ring[b % 4]
            wait_out(nbuf, nosem)
            start_in(i + 2, nbuf, nisem)
        return carry

    lax.fori_loop(1, SPW * 2 // 4, loop_body, jnp.int32(0))

    for b in range(2):
        buf, isem, osem = ring[2 + b]
        i = jnp.int32(SPW * 2 - 2 + b)
        wait_in(buf, isem)
        compute(buf, make_base(i))
        start_out(i, buf, osem)
    for buf, isem, osem in ring:
        wait_out(buf, osem)


@jax.jit
def kernel(x, coefficients_vect, zero_knot_indexes, grid):
    del zero_knot_indexes
    cv2 = coefficients_vect.astype(jnp.float32).reshape(NUM_ACT, SIZE)
    pad = jnp.zeros((NUM_ACT, CSTRIDE - SIZE), jnp.float32)
    ctab = jnp.concatenate([cv2, pad], axis=1).reshape(TABLE_PAD)
    dv2 = jnp.concatenate(
        [cv2[:, 1:] - cv2[:, :-1], jnp.zeros((NUM_ACT, 1), jnp.float32)], axis=1)
    dtab = jnp.concatenate([dv2, pad], axis=1).reshape(TABLE_PAD)
    invg = jnp.broadcast_to(jnp.float32(1.0) / grid[0].astype(jnp.float32), (16,))
    x3 = x.reshape(NSLAB, ROWS, ROWS)

    run = pl.kernel(
        _body,
        out_type=jax.ShapeDtypeStruct((NSLAB, ROWS, ROWS), jnp.float32),
        mesh=plsc.VectorSubcoreMesh(
            core_axis_name="c", subcore_axis_name="s",
            num_cores=NC, num_subcores=NS),
        compiler_params=pltpu.CompilerParams(
            needs_layout_passes=False, use_tc_tiling_on_sc=True),
        scratch_types=[
            pltpu.VMEM((TABLE_PAD,), jnp.float32),
            pltpu.VMEM((TABLE_PAD,), jnp.float32),
            pltpu.VMEM((16,), jnp.float32),
            pltpu.VMEM((HALF, ROWS), jnp.float32),
            pltpu.VMEM((HALF, ROWS), jnp.float32),
            pltpu.VMEM((HALF, ROWS), jnp.float32),
            pltpu.VMEM((HALF, ROWS), jnp.float32),
            pltpu.SemaphoreType.DMA,
            pltpu.SemaphoreType.DMA,
            pltpu.SemaphoreType.DMA,
            pltpu.SemaphoreType.DMA,
            pltpu.SemaphoreType.DMA,
            pltpu.SemaphoreType.DMA,
            pltpu.SemaphoreType.DMA,
            pltpu.SemaphoreType.DMA,
        ],
    )
    out3 = run(x3, ctab, dtab, invg)
    return out3.reshape(x.shape)
